# baseline (speedup 1.0000x reference)
import jax
import jax.numpy as jnp
from jax import lax
from jax.experimental import pallas as pl
from jax.experimental.pallas import tpu as pltpu
from jax.experimental.pallas import tpu_sc as plsc

EMB_DIM = 32
BATCH = 16384
VOCAB = 1000000
SUP = VOCAB // 4
NFULL = 7812
NBLK = 490

_INFO = plsc.get_sparse_core_info()
NC = _INFO.num_cores
NS = _INFO.num_subcores
CPT = BATCH // NS
HALF = CPT // 4
CHUNK = 128
NCHUNK = HALF // CHUNK


def _phase1(tab, tail, packed, s, bufs, obufs, obuf64, sems_in, sems_out):
    lane = lax.iota(jnp.int32, 16)

    def issue_in(i, b):
        j = jnp.minimum(i * 16 + s, NFULL - 1)
        off = pl.multiple_of(j * 128, 128)
        pltpu.async_copy(tab.at[:, pl.ds(off, 128)], bufs[b], sems_in[b])

    def wait_in(b):
        pltpu.make_async_copy(
            tab.at[:, pl.ds(0, 128)], bufs[b], sems_in[b]).wait()

    def issue_out(i, b):
        j = jnp.minimum(i * 16 + s, NFULL - 1)
        off = pl.multiple_of(j * 32, 32)
        pltpu.async_copy(obufs[b], packed.at[pl.ds(off, 32)], sems_out[b])

    def wait_out(b):
        pltpu.make_async_copy(
            obufs[b], packed.at[pl.ds(0, 32)], sems_out[b]).wait()

    def compute(b):
        def mloop(m, carry):
            for h in range(8):
                dv = lane + (16 if h % 2 else 0)
                col = jnp.zeros((16,), jnp.int32) + (4 * m + h // 2)
                vals = plsc.load_gather(bufs[b], [dv, col])
                obufs[b][m, pl.ds(16 * h, 16)] = vals
            return carry
        lax.fori_loop(0, 32, mloop, 0, unroll=4)

    issue_in(jnp.int32(0), 0)

    def pair(g, carry):
        for b in range(2):
            i = g * 2 + b
            wait_in(b)

            @pl.when(i + 1 < NBLK)
            def _():
                issue_in(i + 1, 1 - b)

            @pl.when(i >= 2)
            def _():
                wait_out(b)

            compute(b)
            issue_out(i, b)
        return carry

    lax.fori_loop(0, NBLK // 2, pair, 0)
    wait_out(0)
    wait_out(1)

    @pl.when(s == 4)
    def _():
        pltpu.sync_copy(tail, obuf64)
        pltpu.sync_copy(obuf64, packed.at[pl.ds(NFULL * 32, 16)])


def _phase2(packed, x_hbm, out3, t, s, idx_v, sup, rows128, rows_out, sem_g):
    lane = lax.iota(jnp.int32, 16)
    for half in range(4):
        base = s * CPT + half * HALF
        pltpu.sync_copy(x_hbm.at[pl.ds(base, HALF)], idx_v)

        for c in range(NCHUNK):
            for j in range(CHUNK // 16):
                v = idx_v[pl.ds(c * CHUNK + j * 16, 16)]
                sup[c][pl.ds(j * 16, 16)] = lax.shift_right_logical(v, 2)

        gathers = []
        for c in range(NCHUNK):
            gathers.append(pltpu.async_copy(
                packed.at[sup[c]], rows128.at[pl.ds(c * CHUNK, CHUNK)], sem_g))
        for g in gathers:
            g.wait()

        def extract(g, carry):
            row_idx = g * 16 + lane
            q = idx_v[pl.ds(g * 16, 16)] & 3
            col_base = q * EMB_DIM
            for d in range(EMB_DIM):
                vals = plsc.load_gather(rows128, [row_idx, col_base + d])
                plsc.store_scatter(rows_out, [row_idx, lane * 0 + d], vals)
            return carry

        lax.fori_loop(0, HALF // 16, extract, 0)

        pltpu.sync_copy(rows_out, out3.at[t, pl.ds(base, HALF), :])


def _body(ut, it, xu, xi, tail_u, tail_i, out3, packed_u, packed_i,
          bufs, obufs, obuf64, idx_v, sup, rows128, rows_out,
          sems_in, sems_out, sem_g):
    c = lax.axis_index("c")
    s = lax.axis_index("s")

    @pl.when(c == 0)
    def _():
        _phase1(ut, tail_u, packed_u, s, bufs, obufs, obuf64,
                sems_in, sems_out)

    @pl.when(c == 1)
    def _():
        _phase1(it, tail_i, packed_i, s, bufs, obufs, obuf64,
                sems_in, sems_out)

    plsc.subcore_barrier()

    @pl.when(c == 0)
    def _():
        _phase2(packed_u, xu, out3, 0, s, idx_v, sup, rows128, rows_out,
                sem_g)

    @pl.when(c == 1)
    def _():
        _phase2(packed_i, xi, out3, 1, s, idx_v, sup, rows128, rows_out,
                sem_g)


@jax.jit
def kernel(x, uid_table, iid_table):
    ut = uid_table.T
    it = iid_table.T
    xu = x[:, 0]
    xi = x[:, 1]
    tail_u = uid_table[NFULL * 128:VOCAB].reshape(16, 128)
    tail_i = iid_table[NFULL * 128:VOCAB].reshape(16, 128)
    mesh = plsc.VectorSubcoreMesh(core_axis_name="c", subcore_axis_name="s")
    out3, _, _ = pl.kernel(
        _body,
        out_type=(
            jax.ShapeDtypeStruct((2, BATCH, EMB_DIM), jnp.float32),
            jax.ShapeDtypeStruct((SUP, 128), jnp.float32),
            jax.ShapeDtypeStruct((SUP, 128), jnp.float32),
        ),
        mesh=mesh,
        compiler_params=pltpu.CompilerParams(
            has_side_effects=True, needs_layout_passes=False),
        scratch_types=[
            [pltpu.VMEM((EMB_DIM, 128), jnp.float32) for _ in range(2)],
            [pltpu.VMEM((32, 128), jnp.float32) for _ in range(2)],
            pltpu.VMEM((16, 128), jnp.float32),
            pltpu.VMEM((HALF,), jnp.int32),
            [pltpu.VMEM((CHUNK,), jnp.int32) for _ in range(NCHUNK)],
            pltpu.VMEM((HALF, 128), jnp.float32),
            pltpu.VMEM((HALF, EMB_DIM), jnp.float32),
            [pltpu.SemaphoreType.DMA for _ in range(2)],
            [pltpu.SemaphoreType.DMA for _ in range(2)],
            pltpu.SemaphoreType.DMA,
        ],
    )(ut, it, xu, xi, tail_u, tail_i)
    return out3.transpose(1, 0, 2)


# speedup vs baseline: 1.7990x; 1.7990x over previous
import jax
import jax.numpy as jnp
from jax import lax
from jax.experimental import pallas as pl
from jax.experimental.pallas import tpu as pltpu
from jax.experimental.pallas import tpu_sc as plsc

EMB_DIM = 32
BATCH = 16384
VOCAB = 1000000
SUP = VOCAB // 4
NFULL = 7812
NBLK = 490

_INFO = plsc.get_sparse_core_info()
NC = _INFO.num_cores
NS = _INFO.num_subcores
CPT = BATCH // NS
HALF = CPT // 4
CHUNK = 128
NCHUNK = HALF // CHUNK


def _phase1(tab, tail, packed, s, bufs, obufs, obuf64, sems_in, sems_out):
    lane = lax.iota(jnp.int32, 16)

    def issue_in(i, b):
        j = jnp.minimum(i * 16 + s, NFULL - 1)
        off = pl.multiple_of(j * 128, 128)
        pltpu.async_copy(tab.at[:, pl.ds(off, 128)], bufs[b], sems_in[b])

    def wait_in(b):
        pltpu.make_async_copy(
            tab.at[:, pl.ds(0, 128)], bufs[b], sems_in[b]).wait()

    def issue_out(i, b):
        j = jnp.minimum(i * 16 + s, NFULL - 1)
        off = pl.multiple_of(j * 32, 32)
        pltpu.async_copy(obufs[b], packed.at[pl.ds(off, 32)], sems_out[b])

    def wait_out(b):
        pltpu.make_async_copy(
            obufs[b], packed.at[pl.ds(0, 32)], sems_out[b]).wait()

    def compute(b):
        @plsc.parallel_loop(0, 32, unroll=8)
        def mloop(m):
            for h in range(8):
                dv = lane + (16 if h % 2 else 0)
                col = jnp.zeros((16,), jnp.int32) + (4 * m + h // 2)
                vals = plsc.load_gather(bufs[b], [dv, col])
                obufs[b][m, pl.ds(16 * h, 16)] = vals

    issue_in(jnp.int32(0), 0)

    def pair(g, carry):
        for b in range(2):
            i = g * 2 + b
            wait_in(b)

            @pl.when(i + 1 < NBLK)
            def _():
                issue_in(i + 1, 1 - b)

            @pl.when(i >= 2)
            def _():
                wait_out(b)

            compute(b)
            issue_out(i, b)
        return carry

    lax.fori_loop(0, NBLK // 2, pair, 0)
    wait_out(0)
    wait_out(1)

    @pl.when(s == 4)
    def _():
        pltpu.sync_copy(tail, obuf64)
        pltpu.sync_copy(obuf64, packed.at[pl.ds(NFULL * 32, 16)])


def _phase2(packed, x_hbm, out3, t, s, idx_v, sup, rows128, rows_out, sem_g):
    lane = lax.iota(jnp.int32, 16)
    for half in range(4):
        base = s * CPT + half * HALF
        pltpu.sync_copy(x_hbm.at[pl.ds(base, HALF)], idx_v)

        for c in range(NCHUNK):
            for j in range(CHUNK // 16):
                v = idx_v[pl.ds(c * CHUNK + j * 16, 16)]
                sup[c][pl.ds(j * 16, 16)] = lax.shift_right_logical(v, 2)

        gathers = []
        for c in range(NCHUNK):
            gathers.append(pltpu.async_copy(
                packed.at[sup[c]], rows128.at[pl.ds(c * CHUNK, CHUNK)], sem_g))
        for g in gathers:
            g.wait()

        def extract(g, carry):
            row_idx = g * 16 + lane
            q = idx_v[pl.ds(g * 16, 16)] & 3
            col_base = q * EMB_DIM
            for d in range(EMB_DIM):
                vals = plsc.load_gather(rows128, [row_idx, col_base + d])
                plsc.store_scatter(rows_out, [row_idx, lane * 0 + d], vals)
            return carry

        lax.fori_loop(0, HALF // 16, extract, 0)

        pltpu.sync_copy(rows_out, out3.at[t, pl.ds(base, HALF), :])


def _body(ut, it, xu, xi, tail_u, tail_i, out3, packed_u, packed_i,
          bufs, obufs, obuf64, idx_v, sup, rows128, rows_out,
          sems_in, sems_out, sem_g):
    c = lax.axis_index("c")
    s = lax.axis_index("s")

    @pl.when(c == 0)
    def _():
        _phase1(ut, tail_u, packed_u, s, bufs, obufs, obuf64,
                sems_in, sems_out)

    @pl.when(c == 1)
    def _():
        _phase1(it, tail_i, packed_i, s, bufs, obufs, obuf64,
                sems_in, sems_out)

    plsc.subcore_barrier()

    @pl.when(c == 0)
    def _():
        _phase2(packed_u, xu, out3, 0, s, idx_v, sup, rows128, rows_out,
                sem_g)

    @pl.when(c == 1)
    def _():
        _phase2(packed_i, xi, out3, 1, s, idx_v, sup, rows128, rows_out,
                sem_g)


@jax.jit
def kernel(x, uid_table, iid_table):
    ut = uid_table.T
    it = iid_table.T
    xu = x[:, 0]
    xi = x[:, 1]
    tail_u = uid_table[NFULL * 128:VOCAB].reshape(16, 128)
    tail_i = iid_table[NFULL * 128:VOCAB].reshape(16, 128)
    mesh = plsc.VectorSubcoreMesh(core_axis_name="c", subcore_axis_name="s")
    out3, _, _ = pl.kernel(
        _body,
        out_type=(
            jax.ShapeDtypeStruct((2, BATCH, EMB_DIM), jnp.float32),
            jax.ShapeDtypeStruct((SUP, 128), jnp.float32),
            jax.ShapeDtypeStruct((SUP, 128), jnp.float32),
        ),
        mesh=mesh,
        compiler_params=pltpu.CompilerParams(
            has_side_effects=True, needs_layout_passes=False),
        scratch_types=[
            [pltpu.VMEM((EMB_DIM, 128), jnp.float32) for _ in range(2)],
            [pltpu.VMEM((32, 128), jnp.float32) for _ in range(2)],
            pltpu.VMEM((16, 128), jnp.float32),
            pltpu.VMEM((HALF,), jnp.int32),
            [pltpu.VMEM((CHUNK,), jnp.int32) for _ in range(NCHUNK)],
            pltpu.VMEM((HALF, 128), jnp.float32),
            pltpu.VMEM((HALF, EMB_DIM), jnp.float32),
            [pltpu.SemaphoreType.DMA for _ in range(2)],
            [pltpu.SemaphoreType.DMA for _ in range(2)],
            pltpu.SemaphoreType.DMA,
        ],
    )(ut, it, xu, xi, tail_u, tail_i)
    return out3.transpose(1, 0, 2)
